# baseline (device time: 23708 ns/iter reference)
import jax
import jax.numpy as jnp
from jax import lax
from jax.experimental import pallas as pl
from jax.experimental.pallas import tpu as pltpu

TM = 256


def kernel(x, dy, gamma):
    del gamma
    m, d = x.shape
    n_tiles = m // TM

    def body(x_ref, dy_ref, out_ref, accum, recv1, recv2, sems):
        i = pl.program_id(0)

        @pl.when(i == 0)
        def _():
            accum[...] = jnp.zeros_like(accum)

        xt = x_ref[...]
        dyt = dy_ref[...]
        mu = jnp.mean(xt, axis=1, keepdims=True)
        xc = xt - mu
        var = jnp.mean(xc * xc, axis=1, keepdims=True)
        rstd = lax.rsqrt(var + 1e-5)
        xhat = xc * rstd
        dg = jnp.sum(dyt * xhat, axis=0, keepdims=True)
        db = jnp.sum(dyt, axis=0, keepdims=True)
        accum[...] += jnp.concatenate([dg, db], axis=0)

        @pl.when(i == n_tiles - 1)
        def _():
            my_x = lax.axis_index("x")
            my_y = lax.axis_index("y")
            my_z = lax.axis_index("z")
            p1 = my_z ^ 1
            p2 = my_z ^ 2

            barrier = pltpu.get_barrier_semaphore()
            for pz in (p1, p2):
                pl.semaphore_signal(
                    barrier,
                    inc=1,
                    device_id=(my_x, my_y, pz),
                    device_id_type=pl.DeviceIdType.MESH,
                )
            pl.semaphore_wait(barrier, 2)

            rdma1 = pltpu.make_async_remote_copy(
                src_ref=accum,
                dst_ref=recv1,
                send_sem=sems.at[0],
                recv_sem=sems.at[1],
                device_id=(my_x, my_y, p1),
                device_id_type=pl.DeviceIdType.MESH,
            )
            rdma1.start()
            rdma1.wait()
            accum[...] += recv1[...]

            rdma2 = pltpu.make_async_remote_copy(
                src_ref=accum,
                dst_ref=recv2,
                send_sem=sems.at[2],
                recv_sem=sems.at[3],
                device_id=(my_x, my_y, p2),
                device_id_type=pl.DeviceIdType.MESH,
            )
            rdma2.start()
            rdma2.wait()
            out_ref[...] = accum[...] + recv2[...]

    return pl.pallas_call(
        body,
        grid=(n_tiles,),
        in_specs=[
            pl.BlockSpec((TM, d), lambda i: (i, 0)),
            pl.BlockSpec((TM, d), lambda i: (i, 0)),
        ],
        out_specs=pl.BlockSpec((2, d), lambda i: (0, 0)),
        out_shape=jax.ShapeDtypeStruct((2, d), jnp.float32),
        scratch_shapes=[
            pltpu.VMEM((2, d), jnp.float32),
            pltpu.VMEM((2, d), jnp.float32),
            pltpu.VMEM((2, d), jnp.float32),
            pltpu.SemaphoreType.DMA((4,)),
        ],
        compiler_params=pltpu.CompilerParams(collective_id=0),
    )(x, dy)


# device time: 22619 ns/iter; 1.0481x vs baseline; 1.0481x over previous
import jax
import jax.numpy as jnp
from jax import lax
from jax.experimental import pallas as pl
from jax.experimental.pallas import tpu as pltpu

TM = 256


def kernel(x, dy, gamma):
    del gamma
    m, d = x.shape
    n_tiles = m // TM

    def body(x_ref, dy_ref, out_ref, accum, recvs, ssems, rsems):
        i = pl.program_id(0)
        my_x = lax.axis_index("x")
        my_y = lax.axis_index("y")
        my_z = lax.axis_index("z")

        @pl.when(i == 0)
        def _():
            accum[...] = jnp.zeros_like(accum)
            barrier = pltpu.get_barrier_semaphore()
            for k in (1, 2, 3):
                pl.semaphore_signal(
                    barrier,
                    inc=1,
                    device_id=(my_x, my_y, (my_z + k) % 4),
                    device_id_type=pl.DeviceIdType.MESH,
                )
            pl.semaphore_wait(barrier, 3)

        xt = x_ref[...]
        dyt = dy_ref[...]
        mu = jnp.mean(xt, axis=1, keepdims=True)
        xc = xt - mu
        var = jnp.mean(xc * xc, axis=1, keepdims=True)
        rstd = lax.rsqrt(var + 1e-5)
        xhat = xc * rstd
        dg = jnp.sum(dyt * xhat, axis=0, keepdims=True)
        db = jnp.sum(dyt, axis=0, keepdims=True)
        accum[...] += jnp.concatenate([dg, db], axis=0)

        @pl.when(i == n_tiles - 1)
        def _():
            rdmas = []
            for k in (1, 2, 3):
                rdma = pltpu.make_async_remote_copy(
                    src_ref=accum,
                    dst_ref=recvs.at[k - 1],
                    send_sem=ssems.at[k - 1],
                    recv_sem=rsems.at[k - 1],
                    device_id=(my_x, my_y, (my_z + k) % 4),
                    device_id_type=pl.DeviceIdType.MESH,
                )
                rdma.start()
                rdmas.append(rdma)
            total = accum[...]
            for k, rdma in zip((1, 2, 3), rdmas):
                rdma.wait_send()
                rdma.wait_recv()
                total = total + recvs[k - 1]
            out_ref[...] = total

    return pl.pallas_call(
        body,
        grid=(n_tiles,),
        in_specs=[
            pl.BlockSpec((TM, d), lambda i: (i, 0)),
            pl.BlockSpec((TM, d), lambda i: (i, 0)),
        ],
        out_specs=pl.BlockSpec((2, d), lambda i: (0, 0)),
        out_shape=jax.ShapeDtypeStruct((2, d), jnp.float32),
        scratch_shapes=[
            pltpu.VMEM((2, d), jnp.float32),
            pltpu.VMEM((3, 2, d), jnp.float32),
            pltpu.SemaphoreType.DMA((3,)),
            pltpu.SemaphoreType.DMA((3,)),
        ],
        compiler_params=pltpu.CompilerParams(collective_id=0),
    )(x, dy)
